# Initial kernel scaffold; baseline (speedup 1.0000x reference)
import jax
import jax.numpy as jnp
from jax.experimental import pallas as pl


def kernel(input_tensor):
    raise NotImplementedError("write your pallas kernel here")


# speedup vs baseline: 6.6957x; 6.6957x over previous
import functools

import jax
import jax.numpy as jnp
from jax import lax
from jax.experimental import pallas as pl
from jax.experimental.pallas import tpu as pltpu
from jax.experimental.pallas import tpu_sc as plsc

ROWS = 64
N = 32768
K = 256
L = 16
NV = N // L
NWORKERS = 32
ROWS_PER_WORKER = ROWS // NWORKERS
CAND = N + L


def _mono(f):
  f = f + 0.0
  b = plsc.bitcast(f, jnp.int32)
  return b ^ lax.shift_right_logical(lax.shift_right_arithmetic(b, 31), 1)


def _unmono(m):
  return m ^ lax.shift_right_logical(lax.shift_right_arithmetic(m, 31), 1)


def _bin_top(m):
  return lax.shift_right_logical(m, 24) ^ 0x80


def _bin_at(m, shift):
  return lax.shift_right_logical(m, shift) & 0xFF


def _clear_hist(hist):
  zeros = jnp.zeros((L,), jnp.int32)
  for i in range(256 // L):
    hist[pl.ds(i * L, L)] = zeros


def _find_bin(hist, need):
  lane = lax.iota(jnp.int32, L)

  def body(i, carry):
    carry_sum, found, b_star, s_above = carry
    v = 15 - i
    hv = hist[pl.ds(v * L, L)]
    hr = lax.rev(hv, (0,))
    cs = plsc.cumsum(hr)
    tb = cs + carry_sum
    ok = tb >= need
    r = jnp.min(jnp.where(ok, lane, L))
    hit = r < L
    selm = lane == r
    hbs = jnp.max(jnp.where(selm, hr, 0))
    tbs = jnp.max(jnp.where(selm, tb, 0))
    use = jnp.logical_and(hit, jnp.logical_not(found))
    b_star = jnp.where(use, v * L + 15 - r, b_star)
    s_above = jnp.where(use, tbs - hbs, s_above)
    carry_sum = carry_sum + jnp.max(cs)
    return (carry_sum, jnp.logical_or(found, hit), b_star, s_above)

  init = (jnp.int32(0), jnp.bool_(False), jnp.int32(0), jnp.int32(0))
  _, _, b_star, s_above = lax.fori_loop(0, 256 // L, body, init)
  return b_star, s_above


def _sort_pairs(out_mu, out_idx):
  lane = lax.iota(jnp.int32, L)
  k = 2
  while k <= K:
    j = k // 2
    while j >= 1:
      if j >= L:
        jj = j // L
        kk = k // L

        def pair_body(pi, c, jj=jj, kk=kk):
          va = (pi // jj) * (2 * jj) + (pi % jj)
          vb = va + jj
          amu = out_mu[pl.ds(va * L, L)]
          aid = out_idx[pl.ds(va * L, L)]
          bmu = out_mu[pl.ds(vb * L, L)]
          bid = out_idx[pl.ds(vb * L, L)]
          a_gt = jnp.logical_or(amu > bmu,
                                jnp.logical_and(amu == bmu, aid < bid))
          desc = (va & kk) == 0
          keep = a_gt == desc
          out_mu[pl.ds(va * L, L)] = jnp.where(keep, amu, bmu)
          out_idx[pl.ds(va * L, L)] = jnp.where(keep, aid, bid)
          out_mu[pl.ds(vb * L, L)] = jnp.where(keep, bmu, amu)
          out_idx[pl.ds(vb * L, L)] = jnp.where(keep, bid, aid)
          return c

        lax.fori_loop(0, (K // L) // 2, pair_body, 0)
      else:

        def v_body(v, c, j=j, k=k):
          base = v * L
          p = base + lane
          q = p ^ j
          pmu = plsc.load_gather(out_mu, [q])
          pid = plsc.load_gather(out_idx, [q])
          mmu = out_mu[pl.ds(base, L)]
          mid = out_idx[pl.ds(base, L)]
          my_gt = jnp.logical_or(mmu > pmu,
                                 jnp.logical_and(mmu == pmu, mid < pid))
          desc = (p & k) == 0
          low = (lane & j) == 0
          keep = my_gt == (desc == low)
          out_mu[pl.ds(base, L)] = jnp.where(keep, mmu, pmu)
          out_idx[pl.ds(base, L)] = jnp.where(keep, mid, pid)
          return c

        lax.fori_loop(0, K // L, v_body, 0)
      j //= 2
    k *= 2


def _topk_body(x_hbm, vals_hbm, idx_hbm, row_v, cand_a, cand_b, hist,
               out_mu, out_idx, out_val):
  cid = lax.axis_index("c")
  sid = lax.axis_index("s")
  wid = sid * 2 + cid
  lane = lax.iota(jnp.int32, L)

  def do_row(rr, _):
    row = wid * ROWS_PER_WORKER + rr
    pltpu.sync_copy(x_hbm.at[row], row_v)

    _clear_hist(hist)

    def h0_body(i, c):
      m = _mono(row_v[pl.ds(i * L, L)])
      bv = _bin_top(m)
      cnt, last = plsc.scan_count(bv)
      plsc.addupdate_scatter(hist, [bv], cnt, mask=last)
      return c

    lax.fori_loop(0, NV, h0_body, 0)
    b0, s0 = _find_bin(hist, jnp.int32(K))
    need = jnp.int32(K) - s0

    def c0_body(i, pos):
      m = _mono(row_v[pl.ds(i * L, L)])
      sel = _bin_top(m) == b0
      plsc.store_compressed(cand_a.at[pl.ds(pos, L)], m, mask=sel)
      return pos + jnp.sum(sel.astype(jnp.int32))

    e_cnt = lax.fori_loop(0, NV, c0_body, jnp.int32(0))

    bins = [b0]
    src, dst = cand_a, cand_b
    for lvl in range(1, 4):
      shift = 24 - 8 * lvl
      nv_e = (e_cnt + (L - 1)) // L
      _clear_hist(hist)

      def hl_body(i, c, src=src, shift=shift, e_cnt=e_cnt):
        m = src[pl.ds(i * L, L)]
        valid = (i * L + lane) < e_cnt
        bv = _bin_at(m, shift)
        cnt, last = plsc.scan_count(bv, mask=valid)
        plsc.addupdate_scatter(hist, [bv], cnt, mask=last)
        return c

      lax.fori_loop(0, nv_e, hl_body, 0)
      bl, sl = _find_bin(hist, need)
      need = need - sl
      bins.append(bl)
      if lvl < 3:

        def cl_body(i, pos, src=src, dst=dst, shift=shift, e_cnt=e_cnt, bl=bl):
          m = src[pl.ds(i * L, L)]
          valid = (i * L + lane) < e_cnt
          sel = jnp.logical_and(valid, _bin_at(m, shift) == bl)
          plsc.store_compressed(dst.at[pl.ds(pos, L)], m, mask=sel)
          return pos + jnp.sum(sel.astype(jnp.int32))

        e_cnt = lax.fori_loop(0, nv_e, cl_body, jnp.int32(0))
        src, dst = dst, src

    b0_, b1_, b2_, b3_ = bins
    thresh = (lax.shift_left(b0_ ^ 0x80, 24) | lax.shift_left(b1_, 16)
              | lax.shift_left(b2_, 8) | b3_)
    need_eq = need

    def em_body(i, carry):
      pos, eqc = carry
      m = _mono(row_v[pl.ds(i * L, L)])
      gt = m > thresh
      eq = m == thresh
      ec = plsc.cumsum(eq.astype(jnp.int32))
      take_eq = jnp.logical_and(eq, (ec + eqc) <= need_eq)
      sel = jnp.logical_or(gt, take_eq)
      idxv = i * L + lane
      plsc.store_compressed(out_mu.at[pl.ds(pos, L)], m, mask=sel)
      plsc.store_compressed(out_idx.at[pl.ds(pos, L)], idxv, mask=sel)
      return (pos + jnp.sum(sel.astype(jnp.int32)), eqc + jnp.max(ec))

    lax.fori_loop(0, NV, em_body, (jnp.int32(0), jnp.int32(0)))

    _sort_pairs(out_mu, out_idx)

    def ux_body(i, c):
      m = out_mu[pl.ds(i * L, L)]
      out_val[pl.ds(i * L, L)] = plsc.bitcast(_unmono(m), jnp.float32)
      return c

    lax.fori_loop(0, K // L, ux_body, 0)
    pltpu.sync_copy(out_val.at[pl.ds(0, K)], vals_hbm.at[row])
    pltpu.sync_copy(out_idx.at[pl.ds(0, K)], idx_hbm.at[row])
    return 0

  lax.fori_loop(0, ROWS_PER_WORKER, do_row, 0)


@jax.jit
def kernel(input_tensor):
  mesh = plsc.VectorSubcoreMesh(core_axis_name="c", subcore_axis_name="s")
  f = pl.kernel(
      _topk_body,
      out_type=(
          jax.ShapeDtypeStruct((ROWS, K), jnp.float32),
          jax.ShapeDtypeStruct((ROWS, K), jnp.int32),
      ),
      mesh=mesh,
      compiler_params=pltpu.CompilerParams(needs_layout_passes=False),
      scratch_types=[
          pltpu.VMEM((N,), jnp.float32),
          pltpu.VMEM((CAND,), jnp.int32),
          pltpu.VMEM((CAND,), jnp.int32),
          pltpu.VMEM((256,), jnp.int32),
          pltpu.VMEM((K + L,), jnp.int32),
          pltpu.VMEM((K + L,), jnp.int32),
          pltpu.VMEM((K + L,), jnp.float32),
      ],
  )
  return f(input_tensor)


# speedup vs baseline: 9.0016x; 1.3444x over previous
import jax
import jax.numpy as jnp
from jax import lax
from jax.experimental import pallas as pl
from jax.experimental.pallas import tpu as pltpu
from jax.experimental.pallas import tpu_sc as plsc

ROWS = 64
N = 32768
K = 256
L = 16
NV = N // L
NWORKERS = 32
ROWS_PER_WORKER = ROWS // NWORKERS
CAND = N + L
UH = 4
UC = 2


def _mono(f):
  f = f + 0.0
  b = plsc.bitcast(f, jnp.int32)
  return b ^ lax.shift_right_logical(lax.shift_right_arithmetic(b, 31), 1)


def _unmono(m):
  return m ^ lax.shift_right_logical(lax.shift_right_arithmetic(m, 31), 1)


def _bin_top(m):
  return lax.shift_right_logical(m, 24) ^ 0x80


def _bin_at(m, shift):
  return lax.shift_right_logical(m, shift) & 0xFF


def _clear_hist(hist):
  zeros = jnp.zeros((L,), jnp.int32)
  for i in range(256 // L):
    hist[pl.ds(i * L, L)] = zeros


def _find_bin(hist, need):
  lane = lax.iota(jnp.int32, L)

  def body(i, carry):
    carry_sum, found, b_star, s_above = carry
    v = 15 - i
    hv = hist[pl.ds(v * L, L)]
    hr = lax.rev(hv, (0,))
    cs = plsc.cumsum(hr)
    tb = cs + carry_sum
    ok = tb >= need
    r = jnp.min(jnp.where(ok, lane, L))
    hit = r < L
    selm = lane == r
    hbs = jnp.max(jnp.where(selm, hr, 0))
    tbs = jnp.max(jnp.where(selm, tb, 0))
    use = jnp.logical_and(hit, jnp.logical_not(found))
    b_star = jnp.where(use, v * L + 15 - r, b_star)
    s_above = jnp.where(use, tbs - hbs, s_above)
    carry_sum = carry_sum + jnp.max(cs)
    return (carry_sum, jnp.logical_or(found, hit), b_star, s_above)

  init = (jnp.int32(0), jnp.bool_(False), jnp.int32(0), jnp.int32(0))
  _, _, b_star, s_above = lax.fori_loop(0, 256 // L, body, init)
  return b_star, s_above


def _sort_pairs(out_mu, out_idx):
  lane = lax.iota(jnp.int32, L)
  k = 2
  while k <= K:
    j = k // 2
    while j >= 1:
      if j >= L:
        jj = j // L
        kk = k // L

        def pair_body(pi, c, jj=jj, kk=kk):
          va = (pi // jj) * (2 * jj) + (pi % jj)
          vb = va + jj
          amu = out_mu[pl.ds(va * L, L)]
          aid = out_idx[pl.ds(va * L, L)]
          bmu = out_mu[pl.ds(vb * L, L)]
          bid = out_idx[pl.ds(vb * L, L)]
          a_gt = jnp.logical_or(amu > bmu,
                                jnp.logical_and(amu == bmu, aid < bid))
          desc = (va & kk) == 0
          keep = a_gt == desc
          out_mu[pl.ds(va * L, L)] = jnp.where(keep, amu, bmu)
          out_idx[pl.ds(va * L, L)] = jnp.where(keep, aid, bid)
          out_mu[pl.ds(vb * L, L)] = jnp.where(keep, bmu, amu)
          out_idx[pl.ds(vb * L, L)] = jnp.where(keep, bid, aid)
          return c

        lax.fori_loop(0, (K // L) // 2, pair_body, 0)
      else:

        def v_body(v, c, j=j, k=k):
          base = v * L
          p = base + lane
          q = p ^ j
          pmu = plsc.load_gather(out_mu, [q])
          pid = plsc.load_gather(out_idx, [q])
          mmu = out_mu[pl.ds(base, L)]
          mid = out_idx[pl.ds(base, L)]
          my_gt = jnp.logical_or(mmu > pmu,
                                 jnp.logical_and(mmu == pmu, mid < pid))
          desc = (p & k) == 0
          low = (lane & j) == 0
          keep = my_gt == (desc == low)
          out_mu[pl.ds(base, L)] = jnp.where(keep, mmu, pmu)
          out_idx[pl.ds(base, L)] = jnp.where(keep, mid, pid)
          return c

        lax.fori_loop(0, K // L, v_body, 0)
      j //= 2
    k *= 2


def _topk_body(x_hbm, vals_hbm, idx_hbm, row_v, cand_mu, cand_idx, hist,
               out_mu, out_idx, out_val):
  cid = lax.axis_index("c")
  sid = lax.axis_index("s")
  wid = sid * 2 + cid
  lane = lax.iota(jnp.int32, L)

  def do_row(rr, _):
    row = wid * ROWS_PER_WORKER + rr
    pltpu.sync_copy(x_hbm.at[row], row_v)

    _clear_hist(hist)

    def h0_body(i, c):
      base = i * (UH * L)
      for u in range(UH):
        m = _mono(row_v[pl.ds(base + u * L, L)])
        bv = _bin_top(m)
        cnt, last = plsc.scan_count(bv)
        plsc.addupdate_scatter(hist, [bv], cnt, mask=last)
      return c

    lax.fori_loop(0, NV // UH, h0_body, 0)
    b0, s0 = _find_bin(hist, jnp.int32(K))
    need = jnp.int32(K) - s0

    def c0_body(i, carry):
      pos_a, pos_b = carry
      base = i * (UC * L)
      for u in range(UC):
        m = _mono(row_v[pl.ds(base + u * L, L)])
        bv = _bin_top(m)
        idxv = base + u * L + lane
        sel_a = bv > b0
        sel_b = bv == b0
        plsc.store_compressed(out_mu.at[pl.ds(pos_a, L)], m, mask=sel_a)
        plsc.store_compressed(out_idx.at[pl.ds(pos_a, L)], idxv, mask=sel_a)
        pos_a = pos_a + jnp.sum(sel_a.astype(jnp.int32))
        plsc.store_compressed(cand_mu.at[pl.ds(pos_b, L)], m, mask=sel_b)
        plsc.store_compressed(cand_idx.at[pl.ds(pos_b, L)], idxv, mask=sel_b)
        pos_b = pos_b + jnp.sum(sel_b.astype(jnp.int32))
      return (pos_a, pos_b)

    pos_a, e_cnt = lax.fori_loop(0, NV // UC, c0_body,
                                 (jnp.int32(0), jnp.int32(0)))

    for lvl in range(1, 4):
      shift = 24 - 8 * lvl
      nv_e = (e_cnt + (L - 1)) // L
      _clear_hist(hist)

      def hl_body(i, c, shift=shift, e_cnt=e_cnt):
        m = cand_mu[pl.ds(i * L, L)]
        valid = (i * L + lane) < e_cnt
        bv = _bin_at(m, shift)
        cnt, last = plsc.scan_count(bv, mask=valid)
        plsc.addupdate_scatter(hist, [bv], cnt, mask=last)
        return c

      lax.fori_loop(0, nv_e, hl_body, 0)
      bl, sl = _find_bin(hist, need)
      need = need - sl

      def cl_body(i, carry, shift=shift, e_cnt=e_cnt, bl=bl):
        pos_a, pos_b = carry
        m = cand_mu[pl.ds(i * L, L)]
        iv = cand_idx[pl.ds(i * L, L)]
        valid = (i * L + lane) < e_cnt
        bv = _bin_at(m, shift)
        sel_a = jnp.logical_and(valid, bv > bl)
        sel_b = jnp.logical_and(valid, bv == bl)
        plsc.store_compressed(out_mu.at[pl.ds(pos_a, L)], m, mask=sel_a)
        plsc.store_compressed(out_idx.at[pl.ds(pos_a, L)], iv, mask=sel_a)
        pos_a = pos_a + jnp.sum(sel_a.astype(jnp.int32))
        plsc.store_compressed(cand_mu.at[pl.ds(pos_b, L)], m, mask=sel_b)
        plsc.store_compressed(cand_idx.at[pl.ds(pos_b, L)], iv, mask=sel_b)
        pos_b = pos_b + jnp.sum(sel_b.astype(jnp.int32))
        return (pos_a, pos_b)

      pos_a, e_cnt = lax.fori_loop(0, nv_e, cl_body, (pos_a, jnp.int32(0)))

    def eq_body(i, pos, need=need):
      m = cand_mu[pl.ds(i * L, L)]
      iv = cand_idx[pl.ds(i * L, L)]
      take = (i * L + lane) < need
      plsc.store_compressed(out_mu.at[pl.ds(pos, L)], m, mask=take)
      plsc.store_compressed(out_idx.at[pl.ds(pos, L)], iv, mask=take)
      return pos + jnp.sum(take.astype(jnp.int32))

    lax.fori_loop(0, (need + (L - 1)) // L, eq_body, pos_a)

    _sort_pairs(out_mu, out_idx)

    def ux_body(i, c):
      m = out_mu[pl.ds(i * L, L)]
      out_val[pl.ds(i * L, L)] = plsc.bitcast(_unmono(m), jnp.float32)
      return c

    lax.fori_loop(0, K // L, ux_body, 0)
    pltpu.sync_copy(out_val.at[pl.ds(0, K)], vals_hbm.at[row])
    pltpu.sync_copy(out_idx.at[pl.ds(0, K)], idx_hbm.at[row])
    return 0

  lax.fori_loop(0, ROWS_PER_WORKER, do_row, 0)


@jax.jit
def kernel(input_tensor):
  mesh = plsc.VectorSubcoreMesh(core_axis_name="c", subcore_axis_name="s")
  f = pl.kernel(
      _topk_body,
      out_type=(
          jax.ShapeDtypeStruct((ROWS, K), jnp.float32),
          jax.ShapeDtypeStruct((ROWS, K), jnp.int32),
      ),
      mesh=mesh,
      compiler_params=pltpu.CompilerParams(needs_layout_passes=False),
      scratch_types=[
          pltpu.VMEM((N,), jnp.float32),
          pltpu.VMEM((CAND,), jnp.int32),
          pltpu.VMEM((CAND,), jnp.int32),
          pltpu.VMEM((256,), jnp.int32),
          pltpu.VMEM((K + L,), jnp.int32),
          pltpu.VMEM((K + L,), jnp.int32),
          pltpu.VMEM((K + L,), jnp.float32),
      ],
  )
  return f(input_tensor)


# speedup vs baseline: 17.3570x; 1.9282x over previous
import jax
import jax.numpy as jnp
from jax import lax
from jax.experimental import pallas as pl
from jax.experimental.pallas import tpu as pltpu
from jax.experimental.pallas import tpu_sc as plsc

ROWS = 64
N = 32768
K = 256
L = 16
NV = N // L
NWORKERS = 32
ROWS_PER_WORKER = ROWS // NWORKERS
CAND = N + L
GH = 8
GC = 8


def _mono(f):
  f = f + 0.0
  b = plsc.bitcast(f, jnp.int32)
  return b ^ lax.shift_right_logical(lax.shift_right_arithmetic(b, 31), 1)


def _unmono(m):
  return m ^ lax.shift_right_logical(lax.shift_right_arithmetic(m, 31), 1)


def _bin_top(m):
  return lax.shift_right_logical(m, 24) ^ 0x80


def _bin_at(m, shift):
  return lax.shift_right_logical(m, shift) & 0xFF


def _clear_hist(hist):
  zeros = jnp.zeros((L,), jnp.int32)
  for i in range(256 // L):
    hist[pl.ds(i * L, L)] = zeros


def _find_bin(hist, need):
  lane = lax.iota(jnp.int32, L)

  def body(i, carry):
    carry_sum, found, b_star, s_above = carry
    v = 15 - i
    hv = hist[pl.ds(v * L, L)]
    hr = lax.rev(hv, (0,))
    cs = plsc.cumsum(hr)
    tb = cs + carry_sum
    ok = tb >= need
    r = jnp.min(jnp.where(ok, lane, L))
    hit = r < L
    selm = lane == r
    hbs = jnp.max(jnp.where(selm, hr, 0))
    tbs = jnp.max(jnp.where(selm, tb, 0))
    use = jnp.logical_and(hit, jnp.logical_not(found))
    b_star = jnp.where(use, v * L + 15 - r, b_star)
    s_above = jnp.where(use, tbs - hbs, s_above)
    carry_sum = carry_sum + jnp.max(cs)
    return (carry_sum, jnp.logical_or(found, hit), b_star, s_above)

  init = (jnp.int32(0), jnp.bool_(False), jnp.int32(0), jnp.int32(0))
  _, _, b_star, s_above = lax.fori_loop(0, 256 // L, body, init)
  return b_star, s_above


def _sort_pairs(out_mu, out_idx):
  lane = lax.iota(jnp.int32, L)
  k = 2
  while k <= K:
    j = k // 2
    while j >= 1:
      if j >= L:
        jj = j // L
        kk = k // L

        def pair_body(pi, c, jj=jj, kk=kk):
          va = (pi // jj) * (2 * jj) + (pi % jj)
          vb = va + jj
          amu = out_mu[pl.ds(va * L, L)]
          aid = out_idx[pl.ds(va * L, L)]
          bmu = out_mu[pl.ds(vb * L, L)]
          bid = out_idx[pl.ds(vb * L, L)]
          a_gt = jnp.logical_or(amu > bmu,
                                jnp.logical_and(amu == bmu, aid < bid))
          desc = (va & kk) == 0
          keep = a_gt == desc
          out_mu[pl.ds(va * L, L)] = jnp.where(keep, amu, bmu)
          out_idx[pl.ds(va * L, L)] = jnp.where(keep, aid, bid)
          out_mu[pl.ds(vb * L, L)] = jnp.where(keep, bmu, amu)
          out_idx[pl.ds(vb * L, L)] = jnp.where(keep, bid, aid)
          return c

        lax.fori_loop(0, (K // L) // 2, pair_body, 0)
      else:

        def v_body(v, c, j=j, k=k):
          base = v * L
          p = base + lane
          q = p ^ j
          pmu = plsc.load_gather(out_mu, [q])
          pid = plsc.load_gather(out_idx, [q])
          mmu = out_mu[pl.ds(base, L)]
          mid = out_idx[pl.ds(base, L)]
          my_gt = jnp.logical_or(mmu > pmu,
                                 jnp.logical_and(mmu == pmu, mid < pid))
          desc = (p & k) == 0
          low = (lane & j) == 0
          keep = my_gt == (desc == low)
          out_mu[pl.ds(base, L)] = jnp.where(keep, mmu, pmu)
          out_idx[pl.ds(base, L)] = jnp.where(keep, mid, pid)
          return c

        lax.fori_loop(0, K // L, v_body, 0)
      j //= 2
    k *= 2


def _topk_body(x_hbm, vals_hbm, idx_hbm, row_v, cand_mu, cand_idx, hist,
               hist2d, red_s, out_mu, out_idx, out_val):
  cid = lax.axis_index("c")
  sid = lax.axis_index("s")
  wid = sid * 2 + cid
  lane = lax.iota(jnp.int32, L)

  def do_row(rr, _):
    row = wid * ROWS_PER_WORKER + rr
    pltpu.sync_copy(x_hbm.at[row], row_v)

    zeros = jnp.zeros((L,), jnp.int32)

    def hz_body(i, c):
      base = i * (GH * L)
      for u in range(GH):
        hist2d[pl.ds(base + u * L, L)] = zeros
      return c

    lax.fori_loop(0, 4096 // (GH * L), hz_body, 0)
    ones = jnp.ones((L,), jnp.int32)

    def h0_body(i, c):
      base = i * (GH * L)
      ms = [_mono(row_v[pl.ds(base + u * L, L)]) for u in range(GH)]
      for u in range(GH):
        idx2 = lax.shift_left(_bin_top(ms[u]), 4) + lane
        plsc.addupdate_scatter(hist2d, [idx2], ones)
      return c

    lax.fori_loop(0, NV // GH, h0_body, 0)

    def red_body(cki, c):
      for b in range(L):
        vr = hist2d[pl.ds(cki * 256 + b * L, L)]
        red_s[pl.ds(b * L, L)] = plsc.cumsum(vr)
      tot = plsc.load_gather(red_s, [lane * L + (L - 1)])
      hist[pl.ds(cki * L, L)] = tot
      return c

    lax.fori_loop(0, 16, red_body, 0)
    b0, s0 = _find_bin(hist, jnp.int32(K))
    need = jnp.int32(K) - s0

    def c0_body(i, carry):
      base = i * (GC * L)
      ms = [_mono(row_v[pl.ds(base + u * L, L)]) for u in range(GC)]
      anyv = _bin_top(ms[0]) >= b0
      for u in range(1, GC):
        anyv = jnp.logical_or(anyv, _bin_top(ms[u]) >= b0)
      hit = jnp.max(anyv.astype(jnp.int32)) > 0

      def slow(carry):
        pos_a, pos_b = carry
        for u in range(GC):
          m = ms[u]
          bv = _bin_top(m)
          idxv = base + u * L + lane
          sel_a = bv > b0
          sel_b = bv == b0
          plsc.store_compressed(out_mu.at[pl.ds(pos_a, L)], m, mask=sel_a)
          plsc.store_compressed(out_idx.at[pl.ds(pos_a, L)], idxv, mask=sel_a)
          pos_a = pos_a + jnp.sum(sel_a.astype(jnp.int32))
          plsc.store_compressed(cand_mu.at[pl.ds(pos_b, L)], m, mask=sel_b)
          plsc.store_compressed(cand_idx.at[pl.ds(pos_b, L)], idxv, mask=sel_b)
          pos_b = pos_b + jnp.sum(sel_b.astype(jnp.int32))
        return (pos_a, pos_b)

      return lax.cond(hit, slow, lambda c: c, carry)

    pos_a, e_cnt = lax.fori_loop(0, NV // GC, c0_body,
                                 (jnp.int32(0), jnp.int32(0)))

    for lvl in range(1, 4):
      shift = 24 - 8 * lvl
      nv_e = (e_cnt + (L - 1)) // L
      _clear_hist(hist)

      def hl_body(i, c, shift=shift, e_cnt=e_cnt):
        m = cand_mu[pl.ds(i * L, L)]
        valid = (i * L + lane) < e_cnt
        bv = _bin_at(m, shift)
        cnt, last = plsc.scan_count(bv, mask=valid)
        plsc.addupdate_scatter(hist, [bv], cnt, mask=last)
        return c

      lax.fori_loop(0, nv_e, hl_body, 0)
      bl, sl = _find_bin(hist, need)
      need = need - sl

      def cl_body(i, carry, shift=shift, e_cnt=e_cnt, bl=bl):
        pos_a, pos_b = carry
        m = cand_mu[pl.ds(i * L, L)]
        iv = cand_idx[pl.ds(i * L, L)]
        valid = (i * L + lane) < e_cnt
        bv = _bin_at(m, shift)
        sel_a = jnp.logical_and(valid, bv > bl)
        sel_b = jnp.logical_and(valid, bv == bl)
        plsc.store_compressed(out_mu.at[pl.ds(pos_a, L)], m, mask=sel_a)
        plsc.store_compressed(out_idx.at[pl.ds(pos_a, L)], iv, mask=sel_a)
        pos_a = pos_a + jnp.sum(sel_a.astype(jnp.int32))
        plsc.store_compressed(cand_mu.at[pl.ds(pos_b, L)], m, mask=sel_b)
        plsc.store_compressed(cand_idx.at[pl.ds(pos_b, L)], iv, mask=sel_b)
        pos_b = pos_b + jnp.sum(sel_b.astype(jnp.int32))
        return (pos_a, pos_b)

      pos_a, e_cnt = lax.fori_loop(0, nv_e, cl_body, (pos_a, jnp.int32(0)))

    def eq_body(i, pos, need=need):
      m = cand_mu[pl.ds(i * L, L)]
      iv = cand_idx[pl.ds(i * L, L)]
      take = (i * L + lane) < need
      plsc.store_compressed(out_mu.at[pl.ds(pos, L)], m, mask=take)
      plsc.store_compressed(out_idx.at[pl.ds(pos, L)], iv, mask=take)
      return pos + jnp.sum(take.astype(jnp.int32))

    lax.fori_loop(0, (need + (L - 1)) // L, eq_body, pos_a)

    _sort_pairs(out_mu, out_idx)

    def ux_body(i, c):
      m = out_mu[pl.ds(i * L, L)]
      out_val[pl.ds(i * L, L)] = plsc.bitcast(_unmono(m), jnp.float32)
      return c

    lax.fori_loop(0, K // L, ux_body, 0)
    pltpu.sync_copy(out_val.at[pl.ds(0, K)], vals_hbm.at[row])
    pltpu.sync_copy(out_idx.at[pl.ds(0, K)], idx_hbm.at[row])
    return 0

  lax.fori_loop(0, ROWS_PER_WORKER, do_row, 0)


@jax.jit
def kernel(input_tensor):
  mesh = plsc.VectorSubcoreMesh(core_axis_name="c", subcore_axis_name="s")
  f = pl.kernel(
      _topk_body,
      out_type=(
          jax.ShapeDtypeStruct((ROWS, K), jnp.float32),
          jax.ShapeDtypeStruct((ROWS, K), jnp.int32),
      ),
      mesh=mesh,
      compiler_params=pltpu.CompilerParams(needs_layout_passes=False),
      scratch_types=[
          pltpu.VMEM((N,), jnp.float32),
          pltpu.VMEM((CAND,), jnp.int32),
          pltpu.VMEM((CAND,), jnp.int32),
          pltpu.VMEM((256,), jnp.int32),
          pltpu.VMEM((4096,), jnp.int32),
          pltpu.VMEM((256,), jnp.int32),
          pltpu.VMEM((K + L,), jnp.int32),
          pltpu.VMEM((K + L,), jnp.int32),
          pltpu.VMEM((K + L,), jnp.float32),
      ],
  )
  return f(input_tensor)
